# Initial kernel scaffold; baseline (speedup 1.0000x reference)
import jax
import jax.numpy as jnp
from jax.experimental import pallas as pl


def kernel(token_ids, segment_ids, word_emb, pos_emb, seg_emb, ln_w, ln_b, proj_w, proj_b):
    raise NotImplementedError("write your pallas kernel here")


# speedup vs baseline: 1.4759x; 1.4759x over previous
import functools

import jax
import jax.numpy as jnp
from jax import lax
from jax.experimental import pallas as pl
from jax.experimental.pallas import tpu as pltpu
from jax.experimental.pallas import tpu_sc as plsc

_EPS = 1e-12

_NC = 2
_NS = 16
_NW = _NC * _NS


@functools.cache
def _sc_gather(m, v, e, chunk):
    per_w = m // _NW
    n_chunks = per_w // chunk
    mesh = plsc.VectorSubcoreMesh(core_axis_name="c", subcore_axis_name="s")

    @functools.partial(
        pl.kernel,
        out_type=jax.ShapeDtypeStruct((m, e), jnp.float32),
        mesh=mesh,
        scratch_types=[
            pltpu.VMEM((chunk,), jnp.int32),
            pltpu.VMEM((chunk, e), jnp.float32),
            pltpu.SemaphoreType.DMA,
        ],
    )
    def gather_kernel(table_hbm, idx_hbm, out_hbm, idx_v, rows_v, sem):
        wid = lax.axis_index("s") * _NC + lax.axis_index("c")
        base_w = wid * per_w
        for c in range(n_chunks):
            base = base_w + c * chunk
            pltpu.sync_copy(idx_hbm.at[pl.ds(base, chunk)], idx_v)
            pltpu.async_copy(table_hbm.at[idx_v], rows_v, sem).wait()
            pltpu.sync_copy(rows_v, out_hbm.at[pl.ds(base, chunk)])

    return gather_kernel


def _tc_body(x_ref, pos_ref, segf_ref, sege_ref, lnw_ref, lnb_ref, w_ref,
             b_ref, o_ref):
    x = x_ref[...]
    seg0 = sege_ref[0:1, :]
    seg1 = sege_ref[1:2, :]
    emb = x + pos_ref[...] + seg0 + segf_ref[...] * (seg1 - seg0)
    u = jnp.mean(emb, axis=1, keepdims=True)
    d = emb - u
    var = jnp.mean(d * d, axis=1, keepdims=True)
    o = d * lax.rsqrt(var + _EPS)
    o = o * lnw_ref[...] + lnb_ref[...]
    acc = lax.dot_general(
        o.astype(jnp.bfloat16), w_ref[...],
        (((1,), (1,)), ((), ())),
        preferred_element_type=jnp.float32,
    )
    o_ref[...] = acc + b_ref[...]


def _tc_fused(gathered, pos_emb, segf, seg_emb, lnw2, lnb2, proj_w, projb2, s):
    m, e = gathered.shape
    h = proj_w.shape[0]
    bm = 256
    grid = (m // bm,)
    return pl.pallas_call(
        _tc_body,
        grid=grid,
        in_specs=[
            pl.BlockSpec((bm, e), lambda i: (i, 0)),
            pl.BlockSpec((bm, e), lambda i: (i % (s // bm), 0)),
            pl.BlockSpec((bm, 1), lambda i: (i, 0)),
            pl.BlockSpec((2, e), lambda i: (0, 0)),
            pl.BlockSpec((1, e), lambda i: (0, 0)),
            pl.BlockSpec((1, e), lambda i: (0, 0)),
            pl.BlockSpec((h, e), lambda i: (0, 0)),
            pl.BlockSpec((1, h), lambda i: (0, 0)),
        ],
        out_specs=pl.BlockSpec((bm, h), lambda i: (i, 0)),
        out_shape=jax.ShapeDtypeStruct((m, h), jnp.float32),
        compiler_params=pltpu.CompilerParams(
            dimension_semantics=("arbitrary",),
        ),
    )(gathered, pos_emb, segf, seg_emb, lnw2, lnb2,
      proj_w.astype(jnp.bfloat16), projb2)


def kernel(token_ids, segment_ids, word_emb, pos_emb, seg_emb, ln_w, ln_b,
           proj_w, proj_b):
    b, s = token_ids.shape
    v, e = word_emb.shape
    h = proj_w.shape[0]
    m = b * s
    idx = token_ids.reshape(m).astype(jnp.int32)
    gathered = _sc_gather(m, v, e, 64)(word_emb, idx)
    segf = segment_ids.reshape(m, 1).astype(jnp.float32)
    out = _tc_fused(gathered, pos_emb, segf, seg_emb,
                    ln_w.reshape(1, e), ln_b.reshape(1, e),
                    proj_w, proj_b.reshape(1, h), s)
    return out.reshape(b, s, h)
